# baseline (speedup 1.0000x reference)
import functools

import jax
import jax.numpy as jnp
from jax import lax
from jax.experimental import pallas as pl
from jax.experimental.pallas import tpu as pltpu
from jax.experimental.pallas import tpu_sc as plsc

NC = 2
NS = 16
L = 16
NW = NC * NS

NPAD = 10240

_SC_PARAMS = pltpu.CompilerParams(needs_layout_passes=False)
_SC_MESH = plsc.VectorSubcoreMesh(core_axis_name="c", subcore_axis_name="s")


def _worker(cid, sid):
    return sid * NC + cid


def _zero_acc(acc_ref, n):
    def body(i, _):
        acc_ref[pl.ds(i * L, L)] = jnp.zeros((L,), jnp.float32)
        return 0

    lax.fori_loop(0, n // L, body, 0, unroll=8)


def _pass_a_body(e_per, nnz, coef_hbm, he_hbm, sabs_out, csum_out,
                 coef_v, col_v, acc_s, acc_c, sem):
    wid = _worker(lax.axis_index("c"), lax.axis_index("s"))
    base = wid * e_per
    d1 = pltpu.async_copy(coef_hbm.at[pl.ds(base, e_per)], coef_v, sem)
    d2 = pltpu.async_copy(he_hbm.at[pl.ds(nnz + base, e_per)], col_v, sem)
    _zero_acc(acc_s, NPAD)
    _zero_acc(acc_c, NPAD)
    d1.wait()
    d2.wait()

    def body(i, _):
        c = coef_v[pl.ds(i * L, L)]
        idx = col_v[pl.ds(i * L, L)]
        plsc.addupdate_scatter(acc_s, [idx], jnp.abs(c))
        plsc.addupdate_scatter(acc_c, [idx], c)
        return 0

    lax.fori_loop(0, e_per // L, body, 0, unroll=8)
    pltpu.sync_copy(acc_s, sabs_out.at[wid])
    pltpu.sync_copy(acc_c, csum_out.at[wid])


def _pass_b_body(e_per, nnz, coef_hbm, he_hbm, inv_hbm, rhss_hbm,
                 sv_out, u_out, coef_v, row_v, col_v, inv_v, rhss_v,
                 acc_sv, acc_u, sem):
    wid = _worker(lax.axis_index("c"), lax.axis_index("s"))
    base = wid * e_per
    ds = [pltpu.async_copy(coef_hbm.at[pl.ds(base, e_per)], coef_v, sem),
          pltpu.async_copy(he_hbm.at[pl.ds(base, e_per)], row_v, sem),
          pltpu.async_copy(he_hbm.at[pl.ds(nnz + base, e_per)], col_v, sem),
          pltpu.async_copy(inv_hbm, inv_v, sem),
          pltpu.async_copy(rhss_hbm, rhss_v, sem)]
    _zero_acc(acc_sv, NPAD)
    _zero_acc(acc_u, NPAD)
    for d in ds:
        d.wait()

    def body(i, _):
        c = coef_v[pl.ds(i * L, L)]
        r = row_v[pl.ds(i * L, L)]
        cl = col_v[pl.ds(i * L, L)]
        scaled = c * plsc.load_gather(inv_v, [cl])
        plsc.addupdate_scatter(acc_sv, [r], scaled)
        plsc.addupdate_scatter(acc_u, [r], scaled * plsc.load_gather(rhss_v, [cl]))
        return 0

    lax.fori_loop(0, e_per // L, body, 0, unroll=8)
    pltpu.sync_copy(acc_sv, sv_out.at[wid])
    pltpu.sync_copy(acc_u, u_out.at[wid])


def _pass_c_body(e_per, nnz, coef_hbm, he_hbm, xv0_hbm, v_out,
                 coef_v, row_v, col_v, xv0_v, acc_v, sem):
    wid = _worker(lax.axis_index("c"), lax.axis_index("s"))
    base = wid * e_per
    ds = [pltpu.async_copy(coef_hbm.at[pl.ds(base, e_per)], coef_v, sem),
          pltpu.async_copy(he_hbm.at[pl.ds(base, e_per)], row_v, sem),
          pltpu.async_copy(he_hbm.at[pl.ds(nnz + base, e_per)], col_v, sem),
          pltpu.async_copy(xv0_hbm, xv0_v, sem)]
    _zero_acc(acc_v, NPAD)
    for d in ds:
        d.wait()

    def body(i, _):
        c = coef_v[pl.ds(i * L, L)]
        r = row_v[pl.ds(i * L, L)]
        cl = col_v[pl.ds(i * L, L)]
        plsc.addupdate_scatter(acc_v, [cl], c * plsc.load_gather(xv0_v, [r]))
        return 0

    lax.fori_loop(0, e_per // L, body, 0, unroll=8)
    pltpu.sync_copy(acc_v, v_out.at[wid])


def _make_sc_kernels(e_per, nnz):
    f32 = jnp.float32
    i32 = jnp.int32
    pass_a = pl.kernel(
        functools.partial(_pass_a_body, e_per, nnz),
        out_type=[jax.ShapeDtypeStruct((NW, NPAD), f32)] * 2,
        mesh=_SC_MESH,
        compiler_params=_SC_PARAMS,
        scratch_types=[
            pltpu.VMEM((e_per,), f32),
            pltpu.VMEM((e_per,), i32),
            pltpu.VMEM((NPAD,), f32),
            pltpu.VMEM((NPAD,), f32),
            pltpu.SemaphoreType.DMA,
        ],
        name="gnn_milp_pass_a",
    )
    pass_b = pl.kernel(
        functools.partial(_pass_b_body, e_per, nnz),
        out_type=[jax.ShapeDtypeStruct((NW, NPAD), f32)] * 2,
        mesh=_SC_MESH,
        compiler_params=_SC_PARAMS,
        scratch_types=[
            pltpu.VMEM((e_per,), f32),
            pltpu.VMEM((e_per,), i32),
            pltpu.VMEM((e_per,), i32),
            pltpu.VMEM((NPAD,), f32),
            pltpu.VMEM((NPAD,), f32),
            pltpu.VMEM((NPAD,), f32),
            pltpu.VMEM((NPAD,), f32),
            pltpu.SemaphoreType.DMA,
        ],
        name="gnn_milp_pass_b",
    )
    pass_c = pl.kernel(
        functools.partial(_pass_c_body, e_per, nnz),
        out_type=[jax.ShapeDtypeStruct((NW, NPAD), f32)],
        mesh=_SC_MESH,
        compiler_params=_SC_PARAMS,
        scratch_types=[
            pltpu.VMEM((e_per,), f32),
            pltpu.VMEM((e_per,), i32),
            pltpu.VMEM((e_per,), i32),
            pltpu.VMEM((NPAD,), f32),
            pltpu.VMEM((NPAD,), f32),
            pltpu.SemaphoreType.DMA,
        ],
        name="gnn_milp_pass_c",
    )
    return pass_a, pass_b, pass_c


def _safe_recip(s):
    inv = 1.0 / s
    return jnp.where(jnp.isinf(inv), 0.0, inv)


def _g1_body(sabs_ref, csum_ref, rhs_ref, inv_ref, t2_ref, rhss_ref):
    s = jnp.sum(sabs_ref[...], axis=0, keepdims=True)
    cs = jnp.sum(csum_ref[...], axis=0, keepdims=True)
    inv = _safe_recip(s)
    inv_ref[...] = inv
    t2_ref[...] = inv * cs
    rhss_ref[...] = inv * rhs_ref[...]


def _g2_body(sv_ref, u_ref, invv_ref, xv0_ref):
    s = jnp.sum(sv_ref[...], axis=0, keepdims=True)
    u = jnp.sum(u_ref[...], axis=0, keepdims=True)
    inv = _safe_recip(s)
    invv_ref[...] = inv
    xv0_ref[...] = inv * u


def _g3_body(n, v_ref, inv_ref, t2_ref, rhss_ref, wc_ref, bc_ref, w_ref, b_ref,
             t1_ref, rv_ref):
    v = jnp.sum(v_ref[...], axis=0, keepdims=True)
    t1 = inv_ref[...] * v
    t1_ref[...] = t1
    m1 = jnp.sum(t1) / n
    m2 = jnp.sum(t2_ref[...]) / n
    srhs = jnp.sum(rhss_ref[...])
    w = w_ref[...]
    b = b_ref[...]
    mean_vec = m1 * w + m2 * b
    aggr = lax.dot_general(mean_vec, wc_ref[...],
                           (((1,), (1,)), ((), ()))) + bc_ref[...]
    rv_ref[...] = (srhs * w + b) - aggr


def _outer(s, vec):
    return lax.dot_general(s, vec, (((0,), (0,)), ((), ())),
                           preferred_element_type=jnp.float32)


def _emb_body(n, rhss_ref, w_ref, b_ref, emb_ref):
    emb_ref[...] = _outer(rhss_ref[0:1, :n], w_ref[...]) + b_ref[...]


def _final_body(n, xv0_ref, invv_ref, t1_ref, t2_ref, w_ref, b_ref, rv_ref,
                xvar_ref, xconst_ref):
    w = w_ref[...]
    b = b_ref[...]
    rv = rv_ref[...]
    xvar_ref[...] = (_outer(invv_ref[0:1, :n], rv * w)
                     + _outer(xv0_ref[0:1, :n], w) + (rv * b + b))
    xconst_ref[...] = (_outer(t1_ref[0:1, :n], w)
                      + _outer(t2_ref[0:1, :n], b))


def kernel(hyperedge_index, coef, rhs, W_rhs, b_rhs, W_c, b_c):
    f32 = jnp.float32
    nnz = coef.shape[0]
    n = rhs.shape[0]
    D = W_rhs.shape[0]
    e_per = nnz // NW
    assert nnz % NW == 0 and n <= NPAD

    he = hyperedge_index.astype(jnp.int32).reshape(2 * nnz)
    coef = coef.astype(f32)

    pass_a, pass_b, pass_c = _make_sc_kernels(e_per, nnz)

    sabs_p, csum_p = pass_a(coef, he)
    rhs_pad = jnp.pad(rhs[:, 0].astype(f32), (0, NPAD - n)).reshape(1, NPAD)
    inv_sc, t2, rhs_s = pl.pallas_call(
        _g1_body,
        out_shape=[jax.ShapeDtypeStruct((1, NPAD), f32)] * 3,
    )(sabs_p, csum_p, rhs_pad)

    w = W_rhs[:, 0].astype(f32).reshape(1, D)
    b = b_rhs.astype(f32).reshape(1, D)

    emb = pl.pallas_call(
        functools.partial(_emb_body, n),
        out_shape=jax.ShapeDtypeStruct((n, D), f32),
    )(rhs_s, w, b)

    sv_p, u_p = pass_b(coef, he, inv_sc.reshape(NPAD), rhs_s.reshape(NPAD))
    inv_sv, xv0 = pl.pallas_call(
        _g2_body,
        out_shape=[jax.ShapeDtypeStruct((1, NPAD), f32)] * 2,
    )(sv_p, u_p)

    (v_p,) = pass_c(coef, he, xv0.reshape(NPAD))
    t1, rhs_vec = pl.pallas_call(
        functools.partial(_g3_body, float(n)),
        out_shape=[jax.ShapeDtypeStruct((1, NPAD), f32),
                   jax.ShapeDtypeStruct((1, D), f32)],
    )(v_p, inv_sc, t2, rhs_s, W_c.astype(f32), b_c.astype(f32).reshape(1, D),
      w, b)

    xvar, xconst = pl.pallas_call(
        functools.partial(_final_body, n),
        out_shape=[jax.ShapeDtypeStruct((n, D), f32)] * 2,
    )(xv0, inv_sv, t1, t2, w, b, rhs_vec)

    return (xvar, xconst, emb)


# speedup vs baseline: 1.0041x; 1.0041x over previous
import functools

import jax
import jax.numpy as jnp
from jax import lax
from jax.experimental import pallas as pl
from jax.experimental.pallas import tpu as pltpu
from jax.experimental.pallas import tpu_sc as plsc

NC = 2
NS = 16
L = 16
NW = NC * NS

NPAD = 10240
TAIL = 128

_SC_PARAMS = pltpu.CompilerParams(needs_layout_passes=False)
_SC_MESH = plsc.VectorSubcoreMesh(core_axis_name="c", subcore_axis_name="s")


def _worker(cid, sid):
    return sid * NC + cid


def _zero_acc(acc_ref, n):
    def body(i, _):
        acc_ref[pl.ds(i * L, L)] = jnp.zeros((L,), jnp.float32)
        return 0

    lax.fori_loop(0, n // L, body, 0, unroll=8)


def _stage_edges(e_main, rem, wid, coef_hbm, he_hbm, coef_v, he_v,
                 coef_t, he_t, sem):
    base = wid * e_main
    descs = [pltpu.async_copy(coef_hbm.at[pl.ds(base, e_main)], coef_v, sem),
             pltpu.async_copy(he_hbm.at[:, pl.ds(base, e_main)], he_v, sem)]

    @pl.when(wid < rem)
    def _():
        tbase = NW * e_main + wid * TAIL
        d1 = pltpu.async_copy(coef_hbm.at[pl.ds(tbase, TAIL)], coef_t, sem)
        d2 = pltpu.async_copy(he_hbm.at[:, pl.ds(tbase, TAIL)], he_t, sem)
        d1.wait()
        d2.wait()

    @pl.when(wid >= rem)
    def _():
        for j in range(TAIL // L):
            coef_t[pl.ds(j * L, L)] = jnp.zeros((L,), jnp.float32)
            he_t[0, pl.ds(j * L, L)] = jnp.zeros((L,), jnp.int32)
            he_t[1, pl.ds(j * L, L)] = jnp.zeros((L,), jnp.int32)

    return descs


def _edges_loop(n_iters, coef_ref, he_ref, fn, unroll=4):
    def body(i, _):
        c = coef_ref[pl.ds(i * L, L)]
        r = he_ref[0, pl.ds(i * L, L)]
        cl = he_ref[1, pl.ds(i * L, L)]
        fn(c, r, cl)
        return 0

    lax.fori_loop(0, n_iters, body, 0, unroll=unroll)


def _pass_a_body(e_main, rem, coef_hbm, he_hbm, sabs_out, csum_out,
                 coef_v, he_v, coef_t, he_t, acc_s, acc_c, sem):
    wid = _worker(lax.axis_index("c"), lax.axis_index("s"))
    descs = _stage_edges(e_main, rem, wid, coef_hbm, he_hbm, coef_v, he_v,
                         coef_t, he_t, sem)
    _zero_acc(acc_s, NPAD)
    _zero_acc(acc_c, NPAD)
    for d in descs:
        d.wait()

    def fn(c, r, cl):
        plsc.addupdate_scatter(acc_s, [cl], jnp.abs(c))
        plsc.addupdate_scatter(acc_c, [cl], c)

    _edges_loop(e_main // L, coef_v, he_v, fn)
    _edges_loop(TAIL // L, coef_t, he_t, fn)
    pltpu.sync_copy(acc_s, sabs_out.at[wid])
    pltpu.sync_copy(acc_c, csum_out.at[wid])


def _pass_b_body(e_main, rem, coef_hbm, he_hbm, inv_hbm, rhss_hbm,
                 sv_out, u_out, coef_v, he_v, coef_t, he_t, inv_v, rhss_v,
                 acc_sv, acc_u, sem):
    wid = _worker(lax.axis_index("c"), lax.axis_index("s"))
    descs = _stage_edges(e_main, rem, wid, coef_hbm, he_hbm, coef_v, he_v,
                         coef_t, he_t, sem)
    descs.append(pltpu.async_copy(inv_hbm, inv_v, sem))
    descs.append(pltpu.async_copy(rhss_hbm, rhss_v, sem))
    _zero_acc(acc_sv, NPAD)
    _zero_acc(acc_u, NPAD)
    for d in descs:
        d.wait()

    def fn(c, r, cl):
        scaled = c * plsc.load_gather(inv_v, [cl])
        plsc.addupdate_scatter(acc_sv, [r], scaled)
        plsc.addupdate_scatter(acc_u, [r],
                               scaled * plsc.load_gather(rhss_v, [cl]))

    _edges_loop(e_main // L, coef_v, he_v, fn)
    _edges_loop(TAIL // L, coef_t, he_t, fn)
    pltpu.sync_copy(acc_sv, sv_out.at[wid])
    pltpu.sync_copy(acc_u, u_out.at[wid])


def _pass_c_body(e_main, rem, coef_hbm, he_hbm, xv0_hbm, v_out,
                 coef_v, he_v, coef_t, he_t, xv0_v, acc_v, sem):
    wid = _worker(lax.axis_index("c"), lax.axis_index("s"))
    descs = _stage_edges(e_main, rem, wid, coef_hbm, he_hbm, coef_v, he_v,
                         coef_t, he_t, sem)
    descs.append(pltpu.async_copy(xv0_hbm, xv0_v, sem))
    _zero_acc(acc_v, NPAD)
    for d in descs:
        d.wait()

    def fn(c, r, cl):
        plsc.addupdate_scatter(acc_v, [cl], c * plsc.load_gather(xv0_v, [r]))

    _edges_loop(e_main // L, coef_v, he_v, fn)
    _edges_loop(TAIL // L, coef_t, he_t, fn)
    pltpu.sync_copy(acc_v, v_out.at[wid])


def _make_sc_kernels(e_main, rem):
    f32 = jnp.float32
    i32 = jnp.int32
    edge_scratch = [
        pltpu.VMEM((e_main,), f32),
        pltpu.VMEM((2, e_main), i32),
        pltpu.VMEM((TAIL,), f32),
        pltpu.VMEM((2, TAIL), i32),
    ]
    pass_a = pl.kernel(
        functools.partial(_pass_a_body, e_main, rem),
        out_type=[jax.ShapeDtypeStruct((NW, NPAD), f32)] * 2,
        mesh=_SC_MESH,
        compiler_params=_SC_PARAMS,
        scratch_types=edge_scratch + [
            pltpu.VMEM((NPAD,), f32),
            pltpu.VMEM((NPAD,), f32),
            pltpu.SemaphoreType.DMA,
        ],
        name="gnn_milp_pass_a",
    )
    pass_b = pl.kernel(
        functools.partial(_pass_b_body, e_main, rem),
        out_type=[jax.ShapeDtypeStruct((NW, NPAD), f32)] * 2,
        mesh=_SC_MESH,
        compiler_params=_SC_PARAMS,
        scratch_types=edge_scratch + [
            pltpu.VMEM((NPAD,), f32),
            pltpu.VMEM((NPAD,), f32),
            pltpu.VMEM((NPAD,), f32),
            pltpu.VMEM((NPAD,), f32),
            pltpu.SemaphoreType.DMA,
        ],
        name="gnn_milp_pass_b",
    )
    pass_c = pl.kernel(
        functools.partial(_pass_c_body, e_main, rem),
        out_type=[jax.ShapeDtypeStruct((NW, NPAD), f32)],
        mesh=_SC_MESH,
        compiler_params=_SC_PARAMS,
        scratch_types=edge_scratch + [
            pltpu.VMEM((NPAD,), f32),
            pltpu.VMEM((NPAD,), f32),
            pltpu.SemaphoreType.DMA,
        ],
        name="gnn_milp_pass_c",
    )
    return pass_a, pass_b, pass_c


def _safe_recip(s):
    inv = 1.0 / s
    return jnp.where(jnp.isinf(inv), 0.0, inv)


def _g1_body(sabs_ref, csum_ref, rhs_ref, inv_ref, t2_ref, rhss_ref):
    s = jnp.sum(sabs_ref[...], axis=0, keepdims=True)
    cs = jnp.sum(csum_ref[...], axis=0, keepdims=True)
    inv = _safe_recip(s)
    inv_ref[...] = inv
    t2_ref[...] = inv * cs
    rhss_ref[...] = inv * rhs_ref[...]


def _g2_body(sv_ref, u_ref, invv_ref, xv0_ref):
    s = jnp.sum(sv_ref[...], axis=0, keepdims=True)
    u = jnp.sum(u_ref[...], axis=0, keepdims=True)
    inv = _safe_recip(s)
    invv_ref[...] = inv
    xv0_ref[...] = inv * u


def _g3_body(n, v_ref, inv_ref, t2_ref, rhss_ref, wc_ref, bc_ref, w_ref, b_ref,
             t1_ref, rv_ref):
    v = jnp.sum(v_ref[...], axis=0, keepdims=True)
    t1 = inv_ref[...] * v
    t1_ref[...] = t1
    m1 = jnp.sum(t1) / n
    m2 = jnp.sum(t2_ref[...]) / n
    srhs = jnp.sum(rhss_ref[...])
    w = w_ref[...]
    b = b_ref[...]
    mean_vec = m1 * w + m2 * b
    aggr = lax.dot_general(mean_vec, wc_ref[...],
                           (((1,), (1,)), ((), ()))) + bc_ref[...]
    rv_ref[...] = (srhs * w + b) - aggr


def _outer(s, vec):
    return lax.dot_general(s, vec, (((0,), (0,)), ((), ())),
                           preferred_element_type=jnp.float32)


def _emb_body(n, rhss_ref, w_ref, b_ref, emb_ref):
    emb_ref[...] = _outer(rhss_ref[0:1, :n], w_ref[...]) + b_ref[...]


def _final_body(n, xv0_ref, invv_ref, t1_ref, t2_ref, w_ref, b_ref, rv_ref,
                xvar_ref, xconst_ref):
    w = w_ref[...]
    b = b_ref[...]
    rv = rv_ref[...]
    xvar_ref[...] = (_outer(invv_ref[0:1, :n], rv * w)
                     + _outer(xv0_ref[0:1, :n], w) + (rv * b + b))
    xconst_ref[...] = (_outer(t1_ref[0:1, :n], w)
                      + _outer(t2_ref[0:1, :n], b))


def kernel(hyperedge_index, coef, rhs, W_rhs, b_rhs, W_c, b_c):
    f32 = jnp.float32
    nnz = coef.shape[0]
    n = rhs.shape[0]
    D = W_rhs.shape[0]
    ntiles = nnz // TAIL
    e_main = (ntiles // NW) * TAIL
    rem = ntiles % NW
    assert nnz % TAIL == 0 and rem < NW and n <= NPAD

    he = hyperedge_index.astype(jnp.int32)
    coef = coef.astype(f32)

    pass_a, pass_b, pass_c = _make_sc_kernels(e_main, rem)

    sabs_p, csum_p = pass_a(coef, he)
    rhs_pad = jnp.pad(rhs[:, 0].astype(f32), (0, NPAD - n)).reshape(1, NPAD)
    inv_sc, t2, rhs_s = pl.pallas_call(
        _g1_body,
        out_shape=[jax.ShapeDtypeStruct((1, NPAD), f32)] * 3,
    )(sabs_p, csum_p, rhs_pad)

    w = W_rhs[:, 0].astype(f32).reshape(1, D)
    b = b_rhs.astype(f32).reshape(1, D)

    emb = pl.pallas_call(
        functools.partial(_emb_body, n),
        out_shape=jax.ShapeDtypeStruct((n, D), f32),
    )(rhs_s, w, b)

    sv_p, u_p = pass_b(coef, he, inv_sc.reshape(NPAD), rhs_s.reshape(NPAD))
    inv_sv, xv0 = pl.pallas_call(
        _g2_body,
        out_shape=[jax.ShapeDtypeStruct((1, NPAD), f32)] * 2,
    )(sv_p, u_p)

    (v_p,) = pass_c(coef, he, xv0.reshape(NPAD))
    t1, rhs_vec = pl.pallas_call(
        functools.partial(_g3_body, float(n)),
        out_shape=[jax.ShapeDtypeStruct((1, NPAD), f32),
                   jax.ShapeDtypeStruct((1, D), f32)],
    )(v_p, inv_sc, t2, rhs_s, W_c.astype(f32), b_c.astype(f32).reshape(1, D),
      w, b)

    xvar, xconst = pl.pallas_call(
        functools.partial(_final_body, n),
        out_shape=[jax.ShapeDtypeStruct((n, D), f32)] * 2,
    )(xv0, inv_sv, t1, t2, w, b, rhs_vec)

    return (xvar, xconst, emb)


# speedup vs baseline: 1.0197x; 1.0156x over previous
import functools

import jax
import jax.numpy as jnp
from jax import lax
from jax.experimental import pallas as pl
from jax.experimental.pallas import tpu as pltpu
from jax.experimental.pallas import tpu_sc as plsc

NC = 2
NS = 16
L = 16
NW = NC * NS

NPAD = 10240
TAIL = 128

_SC_PARAMS = pltpu.CompilerParams(needs_layout_passes=False)
_SC_MESH = plsc.VectorSubcoreMesh(core_axis_name="c", subcore_axis_name="s")


def _worker(cid, sid):
    return sid * NC + cid


def _zero_acc(acc_ref, n):
    def body(i, _):
        acc_ref[pl.ds(i * L, L)] = jnp.zeros((L,), jnp.float32)
        return 0

    lax.fori_loop(0, n // L, body, 0, unroll=8)


def _stage_edges(e_main, rem, wid, coef_hbm, he_hbm, coef_v, he_v,
                 coef_t, he_t, sem):
    base = wid * e_main
    descs = [pltpu.async_copy(coef_hbm.at[pl.ds(base, e_main)], coef_v, sem),
             pltpu.async_copy(he_hbm.at[:, pl.ds(base, e_main)], he_v, sem)]

    @pl.when(wid < rem)
    def _():
        tbase = NW * e_main + wid * TAIL
        d1 = pltpu.async_copy(coef_hbm.at[pl.ds(tbase, TAIL)], coef_t, sem)
        d2 = pltpu.async_copy(he_hbm.at[:, pl.ds(tbase, TAIL)], he_t, sem)
        d1.wait()
        d2.wait()

    @pl.when(wid >= rem)
    def _():
        for j in range(TAIL // L):
            coef_t[pl.ds(j * L, L)] = jnp.zeros((L,), jnp.float32)
            he_t[0, pl.ds(j * L, L)] = jnp.zeros((L,), jnp.int32)
            he_t[1, pl.ds(j * L, L)] = jnp.zeros((L,), jnp.int32)

    return descs


def _edges_loop(n_iters, coef_ref, he_ref, fn, need_row=True, unroll=4):
    def body(i, _):
        c = coef_ref[pl.ds(i * L, L)]
        r = he_ref[0, pl.ds(i * L, L)] if need_row else None
        cl = he_ref[1, pl.ds(i * L, L)]
        fn(c, r, cl)
        return 0

    lax.fori_loop(0, n_iters, body, 0, unroll=unroll)


def _pass_a_body(e_main, rem, coef_hbm, he_hbm, sabs_out, csum_out,
                 coef_v, he_v, coef_t, he_t, acc_s, acc_c, sem):
    wid = _worker(lax.axis_index("c"), lax.axis_index("s"))
    descs = _stage_edges(e_main, rem, wid, coef_hbm, he_hbm, coef_v, he_v,
                         coef_t, he_t, sem)
    _zero_acc(acc_s, NPAD)
    _zero_acc(acc_c, NPAD)
    for d in descs:
        d.wait()

    def fn(c, r, cl):
        plsc.addupdate_scatter(acc_s, [cl], jnp.abs(c))
        plsc.addupdate_scatter(acc_c, [cl], c)

    _edges_loop(e_main // L, coef_v, he_v, fn, need_row=False)
    _edges_loop(TAIL // L, coef_t, he_t, fn, need_row=False)
    pltpu.sync_copy(acc_s, sabs_out.at[wid])
    pltpu.sync_copy(acc_c, csum_out.at[wid])


def _pass_b_body(e_main, rem, coef_hbm, he_hbm, inv_hbm, rhss_hbm,
                 sv_out, u_out, coef_v, he_v, coef_t, he_t, inv_v, rhss_v,
                 acc_sv, acc_u, sem):
    wid = _worker(lax.axis_index("c"), lax.axis_index("s"))
    descs = _stage_edges(e_main, rem, wid, coef_hbm, he_hbm, coef_v, he_v,
                         coef_t, he_t, sem)
    descs.append(pltpu.async_copy(inv_hbm, inv_v, sem))
    descs.append(pltpu.async_copy(rhss_hbm, rhss_v, sem))
    _zero_acc(acc_sv, NPAD)
    _zero_acc(acc_u, NPAD)
    for d in descs:
        d.wait()

    def fn(c, r, cl):
        scaled = c * plsc.load_gather(inv_v, [cl])
        plsc.addupdate_scatter(acc_sv, [r], scaled)
        plsc.addupdate_scatter(acc_u, [r],
                               scaled * plsc.load_gather(rhss_v, [cl]))

    _edges_loop(e_main // L, coef_v, he_v, fn)
    _edges_loop(TAIL // L, coef_t, he_t, fn)
    pltpu.sync_copy(acc_sv, sv_out.at[wid])
    pltpu.sync_copy(acc_u, u_out.at[wid])


def _pass_c_body(e_main, rem, coef_hbm, he_hbm, xv0_hbm, v_out,
                 coef_v, he_v, coef_t, he_t, xv0_v, acc_v, sem):
    wid = _worker(lax.axis_index("c"), lax.axis_index("s"))
    descs = _stage_edges(e_main, rem, wid, coef_hbm, he_hbm, coef_v, he_v,
                         coef_t, he_t, sem)
    descs.append(pltpu.async_copy(xv0_hbm, xv0_v, sem))
    _zero_acc(acc_v, NPAD)
    for d in descs:
        d.wait()

    def fn(c, r, cl):
        plsc.addupdate_scatter(acc_v, [cl], c * plsc.load_gather(xv0_v, [r]))

    _edges_loop(e_main // L, coef_v, he_v, fn)
    _edges_loop(TAIL // L, coef_t, he_t, fn)
    pltpu.sync_copy(acc_v, v_out.at[wid])


def _make_sc_kernels(e_main, rem):
    f32 = jnp.float32
    i32 = jnp.int32
    edge_scratch = [
        pltpu.VMEM((e_main,), f32),
        pltpu.VMEM((2, e_main), i32),
        pltpu.VMEM((TAIL,), f32),
        pltpu.VMEM((2, TAIL), i32),
    ]
    pass_a = pl.kernel(
        functools.partial(_pass_a_body, e_main, rem),
        out_type=[jax.ShapeDtypeStruct((NW, NPAD), f32)] * 2,
        mesh=_SC_MESH,
        compiler_params=_SC_PARAMS,
        scratch_types=edge_scratch + [
            pltpu.VMEM((NPAD,), f32),
            pltpu.VMEM((NPAD,), f32),
            pltpu.SemaphoreType.DMA,
        ],
        name="gnn_milp_pass_a",
    )
    pass_b = pl.kernel(
        functools.partial(_pass_b_body, e_main, rem),
        out_type=[jax.ShapeDtypeStruct((NW, NPAD), f32)] * 2,
        mesh=_SC_MESH,
        compiler_params=_SC_PARAMS,
        scratch_types=edge_scratch + [
            pltpu.VMEM((NPAD,), f32),
            pltpu.VMEM((NPAD,), f32),
            pltpu.VMEM((NPAD,), f32),
            pltpu.VMEM((NPAD,), f32),
            pltpu.SemaphoreType.DMA,
        ],
        name="gnn_milp_pass_b",
    )
    pass_c = pl.kernel(
        functools.partial(_pass_c_body, e_main, rem),
        out_type=[jax.ShapeDtypeStruct((NW, NPAD), f32)],
        mesh=_SC_MESH,
        compiler_params=_SC_PARAMS,
        scratch_types=edge_scratch + [
            pltpu.VMEM((NPAD,), f32),
            pltpu.VMEM((NPAD,), f32),
            pltpu.SemaphoreType.DMA,
        ],
        name="gnn_milp_pass_c",
    )
    return pass_a, pass_b, pass_c


def _safe_recip(s):
    inv = 1.0 / s
    return jnp.where(jnp.isinf(inv), 0.0, inv)


def _g1_body(sabs_ref, csum_ref, rhs_ref, inv_ref, t2_ref, rhss_ref):
    s = jnp.sum(sabs_ref[...], axis=0, keepdims=True)
    cs = jnp.sum(csum_ref[...], axis=0, keepdims=True)
    inv = _safe_recip(s)
    inv_ref[...] = inv
    t2_ref[...] = inv * cs
    rhss_ref[...] = inv * rhs_ref[...]


def _g2_body(sv_ref, u_ref, invv_ref, xv0_ref):
    s = jnp.sum(sv_ref[...], axis=0, keepdims=True)
    u = jnp.sum(u_ref[...], axis=0, keepdims=True)
    inv = _safe_recip(s)
    invv_ref[...] = inv
    xv0_ref[...] = inv * u


def _outer(s, vec):
    return lax.dot_general(s, vec, (((0,), (0,)), ((), ())),
                           preferred_element_type=jnp.float32)


def _emb_body(n, rhss_ref, w_ref, b_ref, emb_ref):
    emb_ref[...] = _outer(rhss_ref[0:1, :n], w_ref[...]) + b_ref[...]


def _final_body(n, v_ref, inv_ref, xv0_ref, invv_ref, t2_ref, rhss_ref,
                wc_ref, bc_ref, w_ref, b_ref, xvar_ref, xconst_ref):
    w = w_ref[...]
    b = b_ref[...]
    v = jnp.sum(v_ref[...], axis=0, keepdims=True)
    t1 = inv_ref[...] * v
    m1 = jnp.sum(t1) / n
    m2 = jnp.sum(t2_ref[...]) / n
    srhs = jnp.sum(rhss_ref[...])
    mean_vec = m1 * w + m2 * b
    aggr = lax.dot_general(mean_vec, wc_ref[...],
                           (((1,), (1,)), ((), ()))) + bc_ref[...]
    rv = (srhs * w + b) - aggr
    nn = int(n)
    xvar_ref[...] = (_outer(invv_ref[0:1, :nn], rv * w)
                     + _outer(xv0_ref[0:1, :nn], w) + (rv * b + b))
    xconst_ref[...] = (_outer(t1[0:1, :nn], w)
                      + _outer(t2_ref[0:1, :nn], b))


def kernel(hyperedge_index, coef, rhs, W_rhs, b_rhs, W_c, b_c):
    f32 = jnp.float32
    nnz = coef.shape[0]
    n = rhs.shape[0]
    D = W_rhs.shape[0]
    ntiles = nnz // TAIL
    e_main = (ntiles // NW) * TAIL
    rem = ntiles % NW
    assert nnz % TAIL == 0 and rem < NW and n <= NPAD

    he = hyperedge_index.astype(jnp.int32)
    coef = coef.astype(f32)

    pass_a, pass_b, pass_c = _make_sc_kernels(e_main, rem)

    sabs_p, csum_p = pass_a(coef, he)
    rhs_pad = jnp.pad(rhs[:, 0].astype(f32), (0, NPAD - n)).reshape(1, NPAD)
    inv_sc, t2, rhs_s = pl.pallas_call(
        _g1_body,
        out_shape=[jax.ShapeDtypeStruct((1, NPAD), f32)] * 3,
    )(sabs_p, csum_p, rhs_pad)

    w = W_rhs[:, 0].astype(f32).reshape(1, D)
    b = b_rhs.astype(f32).reshape(1, D)

    emb = pl.pallas_call(
        functools.partial(_emb_body, n),
        out_shape=jax.ShapeDtypeStruct((n, D), f32),
    )(rhs_s, w, b)

    sv_p, u_p = pass_b(coef, he, inv_sc.reshape(NPAD), rhs_s.reshape(NPAD))
    inv_sv, xv0 = pl.pallas_call(
        _g2_body,
        out_shape=[jax.ShapeDtypeStruct((1, NPAD), f32)] * 2,
    )(sv_p, u_p)

    (v_p,) = pass_c(coef, he, xv0.reshape(NPAD))
    xvar, xconst = pl.pallas_call(
        functools.partial(_final_body, float(n)),
        out_shape=[jax.ShapeDtypeStruct((n, D), f32)] * 2,
    )(v_p, inv_sc, xv0, inv_sv, t2, rhs_s, W_c.astype(f32),
      b_c.astype(f32).reshape(1, D), w, b)

    return (xvar, xconst, emb)


# speedup vs baseline: 1.2371x; 1.2131x over previous
import functools

import jax
import jax.numpy as jnp
from jax import lax
from jax.experimental import pallas as pl
from jax.experimental.pallas import tpu as pltpu
from jax.experimental.pallas import tpu_sc as plsc

NC = 2
NS = 16
L = 16
NW = NC * NS

NPAD = 10240
TAIL = 128

_SC_PARAMS = pltpu.CompilerParams(needs_layout_passes=False)
_SC_MESH = plsc.VectorSubcoreMesh(core_axis_name="c", subcore_axis_name="s")


def _worker(cid, sid):
    return sid * NC + cid


def _zero_acc(acc_ref, n):
    @plsc.parallel_loop(0, n // L, unroll=8)
    def _(i):
        acc_ref[pl.ds(i * L, L)] = jnp.zeros((L,), jnp.float32)


def _stage_edges(e_main, rem, wid, coef_hbm, he_hbm, coef_v, he_v,
                 coef_t, he_t, sem):
    base = wid * e_main
    descs = [pltpu.async_copy(coef_hbm.at[pl.ds(base, e_main)], coef_v, sem),
             pltpu.async_copy(he_hbm.at[:, pl.ds(base, e_main)], he_v, sem)]

    @pl.when(wid < rem)
    def _():
        tbase = NW * e_main + wid * TAIL
        d1 = pltpu.async_copy(coef_hbm.at[pl.ds(tbase, TAIL)], coef_t, sem)
        d2 = pltpu.async_copy(he_hbm.at[:, pl.ds(tbase, TAIL)], he_t, sem)
        d1.wait()
        d2.wait()

    @pl.when(wid >= rem)
    def _():
        for j in range(TAIL // L):
            coef_t[pl.ds(j * L, L)] = jnp.zeros((L,), jnp.float32)
            he_t[0, pl.ds(j * L, L)] = jnp.zeros((L,), jnp.int32)
            he_t[1, pl.ds(j * L, L)] = jnp.zeros((L,), jnp.int32)

    return descs


def _edges_loop(n_iters, coef_ref, he_ref, fn, need_row=True, unroll=4):
    @plsc.parallel_loop(0, n_iters, unroll=unroll)
    def _(i):
        c = coef_ref[pl.ds(i * L, L)]
        r = he_ref[0, pl.ds(i * L, L)] if need_row else None
        cl = he_ref[1, pl.ds(i * L, L)]
        fn(c, r, cl)


def _pass_a_body(e_main, rem, coef_hbm, he_hbm, sabs_out, csum_out,
                 coef_v, he_v, coef_t, he_t, acc_s, acc_c, sem):
    wid = _worker(lax.axis_index("c"), lax.axis_index("s"))
    descs = _stage_edges(e_main, rem, wid, coef_hbm, he_hbm, coef_v, he_v,
                         coef_t, he_t, sem)
    _zero_acc(acc_s, NPAD)
    _zero_acc(acc_c, NPAD)
    for d in descs:
        d.wait()

    def fn(c, r, cl):
        plsc.addupdate_scatter(acc_s, [cl], jnp.abs(c))
        plsc.addupdate_scatter(acc_c, [cl], c)

    _edges_loop(e_main // L, coef_v, he_v, fn, need_row=False)
    _edges_loop(TAIL // L, coef_t, he_t, fn, need_row=False)
    pltpu.sync_copy(acc_s, sabs_out.at[wid])
    pltpu.sync_copy(acc_c, csum_out.at[wid])


def _pass_b_body(e_main, rem, coef_hbm, he_hbm, inv_hbm, rhss_hbm,
                 sv_out, u_out, coef_v, he_v, coef_t, he_t, inv_v, rhss_v,
                 acc_sv, acc_u, sem):
    wid = _worker(lax.axis_index("c"), lax.axis_index("s"))
    descs = _stage_edges(e_main, rem, wid, coef_hbm, he_hbm, coef_v, he_v,
                         coef_t, he_t, sem)
    descs.append(pltpu.async_copy(inv_hbm, inv_v, sem))
    descs.append(pltpu.async_copy(rhss_hbm, rhss_v, sem))
    _zero_acc(acc_sv, NPAD)
    _zero_acc(acc_u, NPAD)
    for d in descs:
        d.wait()

    def fn(c, r, cl):
        scaled = c * plsc.load_gather(inv_v, [cl])
        plsc.addupdate_scatter(acc_sv, [r], scaled)
        plsc.addupdate_scatter(acc_u, [r],
                               scaled * plsc.load_gather(rhss_v, [cl]))

    _edges_loop(e_main // L, coef_v, he_v, fn)
    _edges_loop(TAIL // L, coef_t, he_t, fn)
    pltpu.sync_copy(acc_sv, sv_out.at[wid])
    pltpu.sync_copy(acc_u, u_out.at[wid])


def _pass_c_body(e_main, rem, coef_hbm, he_hbm, xv0_hbm, v_out,
                 coef_v, he_v, coef_t, he_t, xv0_v, acc_v, sem):
    wid = _worker(lax.axis_index("c"), lax.axis_index("s"))
    descs = _stage_edges(e_main, rem, wid, coef_hbm, he_hbm, coef_v, he_v,
                         coef_t, he_t, sem)
    descs.append(pltpu.async_copy(xv0_hbm, xv0_v, sem))
    _zero_acc(acc_v, NPAD)
    for d in descs:
        d.wait()

    def fn(c, r, cl):
        plsc.addupdate_scatter(acc_v, [cl], c * plsc.load_gather(xv0_v, [r]))

    _edges_loop(e_main // L, coef_v, he_v, fn)
    _edges_loop(TAIL // L, coef_t, he_t, fn)
    pltpu.sync_copy(acc_v, v_out.at[wid])


def _make_sc_kernels(e_main, rem):
    f32 = jnp.float32
    i32 = jnp.int32
    edge_scratch = [
        pltpu.VMEM((e_main,), f32),
        pltpu.VMEM((2, e_main), i32),
        pltpu.VMEM((TAIL,), f32),
        pltpu.VMEM((2, TAIL), i32),
    ]
    pass_a = pl.kernel(
        functools.partial(_pass_a_body, e_main, rem),
        out_type=[jax.ShapeDtypeStruct((NW, NPAD), f32)] * 2,
        mesh=_SC_MESH,
        compiler_params=_SC_PARAMS,
        scratch_types=edge_scratch + [
            pltpu.VMEM((NPAD,), f32),
            pltpu.VMEM((NPAD,), f32),
            pltpu.SemaphoreType.DMA,
        ],
        name="gnn_milp_pass_a",
    )
    pass_b = pl.kernel(
        functools.partial(_pass_b_body, e_main, rem),
        out_type=[jax.ShapeDtypeStruct((NW, NPAD), f32)] * 2,
        mesh=_SC_MESH,
        compiler_params=_SC_PARAMS,
        scratch_types=edge_scratch + [
            pltpu.VMEM((NPAD,), f32),
            pltpu.VMEM((NPAD,), f32),
            pltpu.VMEM((NPAD,), f32),
            pltpu.VMEM((NPAD,), f32),
            pltpu.SemaphoreType.DMA,
        ],
        name="gnn_milp_pass_b",
    )
    pass_c = pl.kernel(
        functools.partial(_pass_c_body, e_main, rem),
        out_type=[jax.ShapeDtypeStruct((NW, NPAD), f32)],
        mesh=_SC_MESH,
        compiler_params=_SC_PARAMS,
        scratch_types=edge_scratch + [
            pltpu.VMEM((NPAD,), f32),
            pltpu.VMEM((NPAD,), f32),
            pltpu.SemaphoreType.DMA,
        ],
        name="gnn_milp_pass_c",
    )
    return pass_a, pass_b, pass_c


def _safe_recip(s):
    inv = 1.0 / s
    return jnp.where(jnp.isinf(inv), 0.0, inv)


def _g1_body(sabs_ref, csum_ref, rhs_ref, inv_ref, t2_ref, rhss_ref):
    s = jnp.sum(sabs_ref[...], axis=0, keepdims=True)
    cs = jnp.sum(csum_ref[...], axis=0, keepdims=True)
    inv = _safe_recip(s)
    inv_ref[...] = inv
    t2_ref[...] = inv * cs
    rhss_ref[...] = inv * rhs_ref[...]


def _g2_body(sv_ref, u_ref, invv_ref, xv0_ref):
    s = jnp.sum(sv_ref[...], axis=0, keepdims=True)
    u = jnp.sum(u_ref[...], axis=0, keepdims=True)
    inv = _safe_recip(s)
    invv_ref[...] = inv
    xv0_ref[...] = inv * u


def _outer(s, vec):
    return lax.dot_general(s, vec, (((0,), (0,)), ((), ())),
                           preferred_element_type=jnp.float32)


def _emb_body(n, rhss_ref, w_ref, b_ref, emb_ref):
    emb_ref[...] = _outer(rhss_ref[0:1, :n], w_ref[...]) + b_ref[...]


def _final_body(n, v_ref, inv_ref, xv0_ref, invv_ref, t2_ref, rhss_ref,
                wc_ref, bc_ref, w_ref, b_ref, xvar_ref, xconst_ref):
    w = w_ref[...]
    b = b_ref[...]
    v = jnp.sum(v_ref[...], axis=0, keepdims=True)
    t1 = inv_ref[...] * v
    m1 = jnp.sum(t1) / n
    m2 = jnp.sum(t2_ref[...]) / n
    srhs = jnp.sum(rhss_ref[...])
    mean_vec = m1 * w + m2 * b
    aggr = lax.dot_general(mean_vec, wc_ref[...],
                           (((1,), (1,)), ((), ()))) + bc_ref[...]
    rv = (srhs * w + b) - aggr
    nn = int(n)
    xvar_ref[...] = (_outer(invv_ref[0:1, :nn], rv * w)
                     + _outer(xv0_ref[0:1, :nn], w) + (rv * b + b))
    xconst_ref[...] = (_outer(t1[0:1, :nn], w)
                      + _outer(t2_ref[0:1, :nn], b))


def kernel(hyperedge_index, coef, rhs, W_rhs, b_rhs, W_c, b_c):
    f32 = jnp.float32
    nnz = coef.shape[0]
    n = rhs.shape[0]
    D = W_rhs.shape[0]
    ntiles = nnz // TAIL
    e_main = (ntiles // NW) * TAIL
    rem = ntiles % NW
    assert nnz % TAIL == 0 and rem < NW and n <= NPAD

    he = hyperedge_index.astype(jnp.int32)
    coef = coef.astype(f32)

    pass_a, pass_b, pass_c = _make_sc_kernels(e_main, rem)

    sabs_p, csum_p = pass_a(coef, he)
    rhs_pad = jnp.pad(rhs[:, 0].astype(f32), (0, NPAD - n)).reshape(1, NPAD)
    inv_sc, t2, rhs_s = pl.pallas_call(
        _g1_body,
        out_shape=[jax.ShapeDtypeStruct((1, NPAD), f32)] * 3,
    )(sabs_p, csum_p, rhs_pad)

    w = W_rhs[:, 0].astype(f32).reshape(1, D)
    b = b_rhs.astype(f32).reshape(1, D)

    emb = pl.pallas_call(
        functools.partial(_emb_body, n),
        out_shape=jax.ShapeDtypeStruct((n, D), f32),
    )(rhs_s, w, b)

    sv_p, u_p = pass_b(coef, he, inv_sc.reshape(NPAD), rhs_s.reshape(NPAD))
    inv_sv, xv0 = pl.pallas_call(
        _g2_body,
        out_shape=[jax.ShapeDtypeStruct((1, NPAD), f32)] * 2,
    )(sv_p, u_p)

    (v_p,) = pass_c(coef, he, xv0.reshape(NPAD))
    xvar, xconst = pl.pallas_call(
        functools.partial(_final_body, float(n)),
        out_shape=[jax.ShapeDtypeStruct((n, D), f32)] * 2,
    )(v_p, inv_sc, xv0, inv_sv, t2, rhs_s, W_c.astype(f32),
      b_c.astype(f32).reshape(1, D), w, b)

    return (xvar, xconst, emb)
